# Initial kernel scaffold; baseline (speedup 1.0000x reference)
import jax
import jax.numpy as jnp
from jax.experimental import pallas as pl


def kernel(x_location, x_patient, ei_patient_visits_location, ei_location_contains_patient, params1, params2, lin_W, lin_b):
    raise NotImplementedError("write your pallas kernel here")


# speedup vs baseline: 44.5365x; 44.5365x over previous
import jax
import jax.numpy as jnp
from jax.experimental import pallas as pl

_GATES = ('i', 'c', 'o')
_EN_PAT = 'location__contains__patient'


def _fused_fwd(x_ref, w1_ref, b1_ref, w2_ref, b2_ref, w3_ref, b3_ref, out_ref):
    x = x_ref[...]
    d1 = w1_ref.shape[1] // 3
    d2 = w2_ref.shape[1] // 3

    pre1 = jnp.dot(x, w1_ref[...], preferred_element_type=jnp.float32) + b1_ref[...]
    i1 = jax.nn.sigmoid(pre1[:, :d1])
    t1 = jnp.tanh(pre1[:, d1:2 * d1])
    o1 = jax.nn.sigmoid(pre1[:, 2 * d1:])
    h1 = jax.nn.relu(o1 * jnp.tanh(i1 * t1))

    pre2 = jnp.dot(h1, w2_ref[...], preferred_element_type=jnp.float32) + b2_ref[...]
    i2 = jax.nn.sigmoid(pre2[:, :d2])
    t2 = jnp.tanh(pre2[:, d2:2 * d2])
    o2 = jax.nn.sigmoid(pre2[:, 2 * d2:])
    h2 = jax.nn.relu(o2 * jnp.tanh(i2 * t2))

    out_ref[...] = (
        jnp.dot(h2, w3_ref[...], preferred_element_type=jnp.float32) + b3_ref[...]
    )


def _cat_weights(p, nt):
    w = jnp.concatenate([p['W_%s_%s' % (g, nt)] for g in _GATES], axis=1)
    b = jnp.concatenate(
        [p['b_%s_%s' % (g, nt)][0] + p['bl_%s_%s' % (g, _EN_PAT)] for g in _GATES]
    )[None, :]
    return w, b


def kernel(x_location, x_patient, ei_patient_visits_location,
           ei_location_contains_patient, params1, params2, lin_W, lin_b):
    n = x_patient.shape[0]
    w1, b1 = _cat_weights(params1, 'patient')
    w2, b2 = _cat_weights(params2, 'patient')
    b3 = lin_b[None, :]

    blk = 1000
    grid = pl.cdiv(n, blk)
    full = lambda i: (0, 0)

    return pl.pallas_call(
        _fused_fwd,
        grid=(grid,),
        in_specs=[
            pl.BlockSpec((blk, x_patient.shape[1]), lambda i: (i, 0)),
            pl.BlockSpec(w1.shape, full),
            pl.BlockSpec(b1.shape, full),
            pl.BlockSpec(w2.shape, full),
            pl.BlockSpec(b2.shape, full),
            pl.BlockSpec(lin_W.shape, full),
            pl.BlockSpec(b3.shape, full),
        ],
        out_specs=pl.BlockSpec((blk, lin_W.shape[1]), lambda i: (i, 0)),
        out_shape=jax.ShapeDtypeStruct((n, lin_W.shape[1]), jnp.float32),
    )(x_patient, w1, b1, w2, b2, lin_W, b3)


# speedup vs baseline: 53.2685x; 1.1961x over previous
import jax
import jax.numpy as jnp
from jax.experimental import pallas as pl

_GATES = ('i', 'c', 'o')
_EN_PAT = 'location__contains__patient'


def _fused_fwd(x_ref,
               wi1, bi1, li1, wc1, bc1, lc1, wo1, bo1, lo1,
               wi2, bi2, li2, wc2, bc2, lc2, wo2, bo2, lo2,
               w3, b3, out_ref):
    x = x_ref[...]

    def pre(h, w_ref, b_ref, l_ref):
        return (jnp.dot(h, w_ref[...], preferred_element_type=jnp.float32)
                + b_ref[...] + l_ref[...])

    i1 = jax.nn.sigmoid(pre(x, wi1, bi1, li1))
    t1 = jnp.tanh(pre(x, wc1, bc1, lc1))
    o1 = jax.nn.sigmoid(pre(x, wo1, bo1, lo1))
    h1 = jax.nn.relu(o1 * jnp.tanh(i1 * t1))

    i2 = jax.nn.sigmoid(pre(h1, wi2, bi2, li2))
    t2 = jnp.tanh(pre(h1, wc2, bc2, lc2))
    o2 = jax.nn.sigmoid(pre(h1, wo2, bo2, lo2))
    h2 = jax.nn.relu(o2 * jnp.tanh(i2 * t2))

    out_ref[...] = (jnp.dot(h2, w3[...], preferred_element_type=jnp.float32)
                    + b3[...])


def kernel(x_location, x_patient, ei_patient_visits_location,
           ei_location_contains_patient, params1, params2, lin_W, lin_b):
    n, d_in = x_patient.shape

    ops, specs = [], []

    def add(a):
        a = a.reshape((1, -1)) if a.ndim == 1 else a
        ops.append(a)
        specs.append(pl.BlockSpec(a.shape, lambda i: (0, 0)))

    for p in (params1, params2):
        for g in _GATES:
            add(p['W_%s_patient' % g])
            add(p['b_%s_patient' % g])
            add(p['bl_%s_%s' % (g, _EN_PAT)])
    add(lin_W)
    add(lin_b)

    blk = 2000
    return pl.pallas_call(
        _fused_fwd,
        grid=(pl.cdiv(n, blk),),
        in_specs=[pl.BlockSpec((blk, d_in), lambda i: (i, 0))] + specs,
        out_specs=pl.BlockSpec((blk, lin_W.shape[1]), lambda i: (i, 0)),
        out_shape=jax.ShapeDtypeStruct((n, lin_W.shape[1]), jnp.float32),
    )(x_patient, *ops)
